# baseline (speedup 1.0000x reference)
import functools

import jax
import jax.numpy as jnp
from jax import lax
from jax.experimental import pallas as pl
from jax.experimental.pallas import tpu as pltpu
from jax.experimental.pallas import tpu_sc as plsc

Z = 64
V = 100000
B = 1024
S = 200
K = 5
NEG_INF = -1000000.0
UNK = 1

VCHUNK = 1000
NBLK = V // VCHUNK

POS = B * S
ROWS = POS * K

try:
    _info = plsc.get_sparse_core_info()
    NC, NS = int(_info.num_cores), int(_info.num_subcores)
except Exception:
    NC, NS = 2, 16
W = NC * NS
IDXW = 128
CH_ROWS = 1280
GPC = CH_ROWS // IDXW
NCHUNK = ROWS // (W * CH_ROWS)
G_MAJOR = ROWS // IDXW
G_PER_W = G_MAJOR // W


def _lse_body(t2_ref, t3_ref, t4_ref, out_ref, m_ref, s_ref):
    i = pl.program_id(0)
    last = pl.num_programs(0) - 1

    @pl.when(i == 0)
    def _init():
        m_ref[...] = jnp.zeros_like(m_ref)
        s_ref[...] = jnp.zeros_like(s_ref)

    for n, blk in enumerate((t2_ref, t3_ref, t4_ref)):
        x = blk[...]
        bm = jnp.max(x, axis=0, keepdims=True)
        bs = jnp.sum(jnp.exp(x - bm), axis=0, keepdims=True)
        m0 = m_ref[n:n + 1, :]
        s0 = s_ref[n:n + 1, :]
        mn = jnp.maximum(m0, bm)
        m_ref[n:n + 1, :] = mn
        s_ref[n:n + 1, :] = s0 * jnp.exp(m0 - mn) + bs * jnp.exp(bm - mn)

    @pl.when(i == last)
    def _fin():
        out_ref[...] = m_ref[...] + jnp.log(s_ref[...])


def _lse_call(e2, e3, e4):
    return pl.pallas_call(
        _lse_body,
        grid=(NBLK,),
        in_specs=[pl.BlockSpec((VCHUNK, Z), lambda i: (i, 0))] * 3,
        out_specs=pl.BlockSpec((8, Z), lambda i: (0, 0)),
        out_shape=jax.ShapeDtypeStruct((8, Z), jnp.float32),
        scratch_shapes=[pltpu.VMEM((8, Z), jnp.float32),
                        pltpu.VMEM((8, Z), jnp.float32)],
    )(e2, e3, e4)


def _build_body(t2_ref, t3_ref, t4_ref, lse_ref, out_ref):
    i = pl.program_id(0)
    rows = lax.broadcasted_iota(jnp.int32, (VCHUNK, 1), 0) + i * VCHUNK
    for n, blk in enumerate((t2_ref, t3_ref, t4_ref)):
        v = blk[...] - lse_ref[n:n + 1, :]
        v = jnp.where(rows == 0, 0.0, v)
        v = jnp.where(rows == 1, NEG_INF, v)
        out_ref[n, :, :] = v


def _build_call(e2, e3, e4, lse):
    return pl.pallas_call(
        _build_body,
        grid=(NBLK,),
        in_specs=[pl.BlockSpec((VCHUNK, Z), lambda i: (i, 0))] * 3
        + [pl.BlockSpec((8, Z), lambda i: (0, 0))],
        out_specs=pl.BlockSpec((3, VCHUNK, Z), lambda i: (0, i, 0)),
        out_shape=jax.ShapeDtypeStruct((3, V, Z), jnp.float32),
    )(e2, e3, e4, lse)


def _sc_body(tall_ref, g_ref, out_ref, idx_v, buf_v, sem):
    wid = lax.axis_index("s") * NC + lax.axis_index("c")
    pltpu.sync_copy(g_ref.at[wid], idx_v)

    def chunk(c, carry):
        copies = [
            pltpu.async_copy(
                tall_ref.at[idx_v.at[c * GPC + j]],
                buf_v.at[pl.ds(j * IDXW, IDXW)],
                sem,
            )
            for j in range(GPC)
        ]
        for cp in copies:
            cp.wait()
        obase = wid * (NCHUNK * CH_ROWS) + c * CH_ROWS
        pltpu.sync_copy(buf_v, out_ref.at[pl.ds(obase, CH_ROWS)])
        return carry

    lax.fori_loop(0, NCHUNK, chunk, 0)


def _sc_call(tall_flat, g3):
    mesh = plsc.VectorSubcoreMesh(core_axis_name="c", subcore_axis_name="s")
    fn = pl.kernel(
        _sc_body,
        out_type=jax.ShapeDtypeStruct((ROWS, Z), jnp.float32),
        mesh=mesh,
        scratch_types=[
            pltpu.VMEM((G_PER_W, IDXW), jnp.int32),
            pltpu.VMEM((CH_ROWS, Z), jnp.float32),
            pltpu.SemaphoreType.DMA,
        ],
        compiler_params=pltpu.CompilerParams(use_tc_tiling_on_sc=False),
    )
    return fn(tall_flat, g3)


def kernel(x, x_lengths, subseq_ids_2, subseq_ids_3, subseq_ids_4,
           transition_matrix_z_z, length_emission_matrix_z_n,
           emission_table_2, emission_table_3, emission_table_4):
    lse = _lse_call(emission_table_2, emission_table_3, emission_table_4)
    tall = _build_call(emission_table_2, emission_table_3, emission_table_4, lse)
    tall_flat = tall.reshape(3 * V, Z)

    def clamp(ids):
        ids = ids.reshape(-1).astype(jnp.int32)
        return jnp.where(ids >= V, UNK, ids)

    i2 = clamp(subseq_ids_2)
    i3 = clamp(subseq_ids_3) + V
    i4 = clamp(subseq_ids_4) + 2 * V
    zero = jnp.zeros((POS,), jnp.int32)
    g3 = jnp.stack([zero, zero, i2, i3, i4], axis=1).reshape(W, G_PER_W, IDXW)

    out = _sc_call(tall_flat, g3)
    return out.reshape(B, S, K, Z)


# speedup vs baseline: 6.0017x; 6.0017x over previous
import functools

import jax
import jax.numpy as jnp
from jax import lax
from jax.experimental import pallas as pl
from jax.experimental.pallas import tpu as pltpu
from jax.experimental.pallas import tpu_sc as plsc

Z = 64
V = 100000
B = 1024
S = 200
K = 5
NEG_INF = -1000000.0
UNK = 1

VCHUNK = 1000
NBLK = V // VCHUNK

PADR = 4000
TP = V + PADR
NBLK_T = TP // VCHUNK

POS = B * S
ROWS = POS * K

try:
    _info = plsc.get_sparse_core_info()
    NC, NS = int(_info.num_cores), int(_info.num_subcores)
except Exception:
    NC, NS = 2, 16
W = NC * NS
IDXW = 128
CH_ROWS = 1280
GPC = CH_ROWS // IDXW
NCHUNK = ROWS // (W * CH_ROWS)
G_MAJOR = ROWS // IDXW
G_PER_W = G_MAJOR // W


def _lse_body(t2_ref, t3_ref, t4_ref, out_ref, m_ref, s_ref):
    i = pl.program_id(0)
    last = pl.num_programs(0) - 1

    @pl.when(i == 0)
    def _init():
        m_ref[...] = jnp.zeros_like(m_ref)
        s_ref[...] = jnp.zeros_like(s_ref)

    for n, blk in enumerate((t2_ref, t3_ref, t4_ref)):
        x = blk[...]
        bm = jnp.max(x, axis=0, keepdims=True)
        bs = jnp.sum(jnp.exp(x - bm), axis=0, keepdims=True)
        m0 = m_ref[n:n + 1, :]
        s0 = s_ref[n:n + 1, :]
        mn = jnp.maximum(m0, bm)
        m_ref[n:n + 1, :] = mn
        s_ref[n:n + 1, :] = s0 * jnp.exp(m0 - mn) + bs * jnp.exp(bm - mn)

    @pl.when(i == last)
    def _fin():
        out_ref[...] = m_ref[...] + jnp.log(s_ref[...])


def _lse_call(e2, e3, e4):
    return pl.pallas_call(
        _lse_body,
        grid=(NBLK,),
        in_specs=[pl.BlockSpec((VCHUNK, Z), lambda i: (i, 0))] * 3,
        out_specs=pl.BlockSpec((8, Z), lambda i: (0, 0)),
        out_shape=jax.ShapeDtypeStruct((8, Z), jnp.float32),
        scratch_shapes=[pltpu.VMEM((8, Z), jnp.float32),
                        pltpu.VMEM((8, Z), jnp.float32)],
    )(e2, e3, e4)


def _build_body(t2_ref, t3_ref, t4_ref, lse_ref, out_ref):
    i = pl.program_id(0)
    rows = lax.broadcasted_iota(jnp.int32, (VCHUNK, 1), 0) + i * VCHUNK
    for n, blk in enumerate((t2_ref, t3_ref, t4_ref)):
        v = blk[...] - lse_ref[n:n + 1, :]
        v = jnp.where(rows == 0, 0.0, v)
        v = jnp.where(rows == 1, NEG_INF, v)
        v = jnp.where(rows >= V, 0.0, v)
        out_ref[n, :, :] = v


def _build_call(e2, e3, e4, lse):
    return pl.pallas_call(
        _build_body,
        grid=(NBLK_T,),
        in_specs=[pl.BlockSpec((VCHUNK, Z), lambda i: (jnp.minimum(i, NBLK - 1), 0))] * 3
        + [pl.BlockSpec((8, Z), lambda i: (0, 0))],
        out_specs=pl.BlockSpec((3, VCHUNK, Z), lambda i: (0, i, 0)),
        out_shape=jax.ShapeDtypeStruct((3, TP, Z), jnp.float32),
    )(e2, e3, e4, lse)


def _sc_body(tall_ref, g_ref, out_ref, idx_v, buf_v, sem):
    wid = lax.axis_index("s") * NC + lax.axis_index("c")
    pltpu.sync_copy(g_ref.at[wid], idx_v)

    def chunk(c, carry):
        copies = [
            pltpu.async_copy(
                tall_ref.at[idx_v.at[c * GPC + j]],
                buf_v.at[pl.ds(j * IDXW, IDXW)],
                sem,
            )
            for j in range(GPC)
        ]
        for cp in copies:
            cp.wait()
        obase = wid * (NCHUNK * CH_ROWS) + c * CH_ROWS
        pltpu.sync_copy(buf_v, out_ref.at[pl.ds(obase, CH_ROWS)])
        return carry

    lax.fori_loop(0, NCHUNK, chunk, 0)


def _sc_call(tall_flat, g3):
    mesh = plsc.VectorSubcoreMesh(core_axis_name="c", subcore_axis_name="s")
    fn = pl.kernel(
        _sc_body,
        out_type=jax.ShapeDtypeStruct((ROWS, Z), jnp.float32),
        mesh=mesh,
        scratch_types=[
            pltpu.VMEM((G_PER_W, IDXW), jnp.int32),
            pltpu.VMEM((CH_ROWS, Z), jnp.float32),
            pltpu.SemaphoreType.DMA,
        ],
        compiler_params=pltpu.CompilerParams(use_tc_tiling_on_sc=False),
    )
    return fn(tall_flat, g3)


def kernel(x, x_lengths, subseq_ids_2, subseq_ids_3, subseq_ids_4,
           transition_matrix_z_z, length_emission_matrix_z_n,
           emission_table_2, emission_table_3, emission_table_4):
    lse = _lse_call(emission_table_2, emission_table_3, emission_table_4)
    tall = _build_call(emission_table_2, emission_table_3, emission_table_4, lse)
    tall_flat = tall.reshape(3 * TP, Z)

    def clamp(ids):
        ids = ids.reshape(-1).astype(jnp.int32)
        return jnp.where(ids >= V, UNK, ids)

    i2 = clamp(subseq_ids_2)
    i3 = clamp(subseq_ids_3) + TP
    i4 = clamp(subseq_ids_4) + 2 * TP
    pos = jnp.arange(POS, dtype=jnp.int32)
    z0 = (pos % 3) * TP + V + (2 * pos) % PADR
    z1 = (pos % 3) * TP + V + (2 * pos + 1) % PADR
    g3 = jnp.stack([z0, z1, i2, i3, i4], axis=1).reshape(W, G_PER_W, IDXW)

    out = _sc_call(tall_flat, g3)
    return out.reshape(B, S, K, Z)


# speedup vs baseline: 6.0393x; 1.0063x over previous
import functools

import jax
import jax.numpy as jnp
from jax import lax
from jax.experimental import pallas as pl
from jax.experimental.pallas import tpu as pltpu
from jax.experimental.pallas import tpu_sc as plsc

Z = 64
V = 100000
B = 1024
S = 200
K = 5
NEG_INF = -1000000.0
UNK = 1

VCHUNK = 1000
NBLK = V // VCHUNK

PADR = 4000
TP = V + PADR
NBLK_T = TP // VCHUNK

POS = B * S
ROWS = POS * K

try:
    _info = plsc.get_sparse_core_info()
    NC, NS = int(_info.num_cores), int(_info.num_subcores)
except Exception:
    NC, NS = 2, 16
W = NC * NS
IDXW = 128
CH_ROWS = 640
NCHUNK = ROWS // (W * CH_ROWS)
PW_ROWS = ROWS // W
G_PER_W = PW_ROWS // IDXW


def _lse_body(t2_ref, t3_ref, t4_ref, out_ref, m_ref, s_ref):
    i = pl.program_id(0)
    last = pl.num_programs(0) - 1

    @pl.when(i == 0)
    def _init():
        m_ref[...] = jnp.zeros_like(m_ref)
        s_ref[...] = jnp.zeros_like(s_ref)

    for n, blk in enumerate((t2_ref, t3_ref, t4_ref)):
        x = blk[...]
        bm = jnp.max(x, axis=0, keepdims=True)
        bs = jnp.sum(jnp.exp(x - bm), axis=0, keepdims=True)
        m0 = m_ref[n:n + 1, :]
        s0 = s_ref[n:n + 1, :]
        mn = jnp.maximum(m0, bm)
        m_ref[n:n + 1, :] = mn
        s_ref[n:n + 1, :] = s0 * jnp.exp(m0 - mn) + bs * jnp.exp(bm - mn)

    @pl.when(i == last)
    def _fin():
        out_ref[...] = m_ref[...] + jnp.log(s_ref[...])


def _lse_call(e2, e3, e4):
    return pl.pallas_call(
        _lse_body,
        grid=(NBLK,),
        in_specs=[pl.BlockSpec((VCHUNK, Z), lambda i: (i, 0))] * 3,
        out_specs=pl.BlockSpec((8, Z), lambda i: (0, 0)),
        out_shape=jax.ShapeDtypeStruct((8, Z), jnp.float32),
        scratch_shapes=[pltpu.VMEM((8, Z), jnp.float32),
                        pltpu.VMEM((8, Z), jnp.float32)],
    )(e2, e3, e4)


def _build_body(t2_ref, t3_ref, t4_ref, lse_ref, out_ref):
    i = pl.program_id(0)
    rows = lax.broadcasted_iota(jnp.int32, (VCHUNK, 1), 0) + i * VCHUNK
    for n, blk in enumerate((t2_ref, t3_ref, t4_ref)):
        v = blk[...] - lse_ref[n:n + 1, :]
        v = jnp.where(rows == 0, 0.0, v)
        v = jnp.where(rows == 1, NEG_INF, v)
        v = jnp.where(rows >= V, 0.0, v)
        out_ref[n, :, :] = v


def _build_call(e2, e3, e4, lse):
    return pl.pallas_call(
        _build_body,
        grid=(NBLK_T,),
        in_specs=[pl.BlockSpec((VCHUNK, Z), lambda i: (jnp.minimum(i, NBLK - 1), 0))] * 3
        + [pl.BlockSpec((8, Z), lambda i: (0, 0))],
        out_specs=pl.BlockSpec((3, VCHUNK, Z), lambda i: (0, i, 0)),
        out_shape=jax.ShapeDtypeStruct((3, TP, Z), jnp.float32),
    )(e2, e3, e4, lse)


def _sc_body(tall_ref, g_ref, out_ref, idx_v, buf0, buf1, sem0, sem1):
    wid = lax.axis_index("s") * NC + lax.axis_index("c")
    pltpu.sync_copy(g_ref.at[wid], idx_v)
    obase = wid * PW_ROWS

    def gather(c, buf, sem):
        return pltpu.make_async_copy(
            tall_ref.at[idx_v.at[pl.ds(c * CH_ROWS, CH_ROWS)]], buf, sem)

    def write(c, buf):
        pltpu.sync_copy(buf, out_ref.at[pl.ds(obase + c * CH_ROWS, CH_ROWS)])

    gather(0, buf0, sem0).start()

    def pair(k, carry):
        c0 = 2 * k
        c1 = c0 + 1
        gather(c1, buf1, sem1).start()
        gather(c0, buf0, sem0).wait()
        write(c0, buf0)

        @pl.when(c0 + 2 < NCHUNK)
        def _():
            gather(c0 + 2, buf0, sem0).start()

        gather(c1, buf1, sem1).wait()
        write(c1, buf1)
        return carry

    lax.fori_loop(0, NCHUNK // 2, pair, 0)


def _sc_call(tall_flat, g3):
    mesh = plsc.VectorSubcoreMesh(core_axis_name="c", subcore_axis_name="s")
    fn = pl.kernel(
        _sc_body,
        out_type=jax.ShapeDtypeStruct((ROWS, Z), jnp.float32),
        mesh=mesh,
        scratch_types=[
            pltpu.VMEM((PW_ROWS,), jnp.int32),
            pltpu.VMEM((CH_ROWS, Z), jnp.float32),
            pltpu.VMEM((CH_ROWS, Z), jnp.float32),
            pltpu.SemaphoreType.DMA,
            pltpu.SemaphoreType.DMA,
        ],
        compiler_params=pltpu.CompilerParams(use_tc_tiling_on_sc=False),
    )
    return fn(tall_flat, g3)


def kernel(x, x_lengths, subseq_ids_2, subseq_ids_3, subseq_ids_4,
           transition_matrix_z_z, length_emission_matrix_z_n,
           emission_table_2, emission_table_3, emission_table_4):
    lse = _lse_call(emission_table_2, emission_table_3, emission_table_4)
    tall = _build_call(emission_table_2, emission_table_3, emission_table_4, lse)
    tall_flat = tall.reshape(3 * TP, Z)

    def clamp(ids):
        ids = ids.reshape(-1).astype(jnp.int32)
        return jnp.where(ids >= V, UNK, ids)

    i2 = clamp(subseq_ids_2)
    i3 = clamp(subseq_ids_3) + TP
    i4 = clamp(subseq_ids_4) + 2 * TP
    pos = jnp.arange(POS, dtype=jnp.int32)
    z0 = (pos % 3) * TP + V + (2 * pos) % PADR
    z1 = (pos % 3) * TP + V + (2 * pos + 1) % PADR
    g3 = jnp.stack([z0, z1, i2, i3, i4], axis=1).reshape(W, PW_ROWS)

    out = _sc_call(tall_flat, g3)
    return out.reshape(B, S, K, Z)


# speedup vs baseline: 6.5215x; 1.0798x over previous
import jax
import jax.numpy as jnp
from jax import lax
from jax.experimental import pallas as pl
from jax.experimental.pallas import tpu as pltpu
from jax.experimental.pallas import tpu_sc as plsc

Z = 64
V = 100000
B = 1024
S = 200
K = 5
NEG_INF = -1000000.0
UNK = 1

VCHUNK = 1000
NBLK = V // VCHUNK

NSLOT = 3
ROWS = S * NSLOT * B

try:
    _info = plsc.get_sparse_core_info()
    NC, NS = int(_info.num_cores), int(_info.num_subcores)
except Exception:
    NC, NS = 2, 16
W = NC * NS
CH_ROWS = 640
PW_ROWS = ROWS // W
NCHUNK = PW_ROWS // CH_ROWS


def _lse_body(t2_ref, t3_ref, t4_ref, out_ref, m_ref, s_ref):
    i = pl.program_id(0)
    last = pl.num_programs(0) - 1

    @pl.when(i == 0)
    def _init():
        m_ref[...] = jnp.zeros_like(m_ref)
        s_ref[...] = jnp.zeros_like(s_ref)

    for n, blk in enumerate((t2_ref, t3_ref, t4_ref)):
        x = blk[...]
        bm = jnp.max(x, axis=0, keepdims=True)
        bs = jnp.sum(jnp.exp(x - bm), axis=0, keepdims=True)
        m0 = m_ref[n:n + 1, :]
        s0 = s_ref[n:n + 1, :]
        mn = jnp.maximum(m0, bm)
        m_ref[n:n + 1, :] = mn
        s_ref[n:n + 1, :] = s0 * jnp.exp(m0 - mn) + bs * jnp.exp(bm - mn)

    @pl.when(i == last)
    def _fin():
        out_ref[...] = m_ref[...] + jnp.log(s_ref[...])


def _lse_call(e2, e3, e4):
    return pl.pallas_call(
        _lse_body,
        grid=(NBLK,),
        in_specs=[pl.BlockSpec((VCHUNK, Z), lambda i: (i, 0))] * 3,
        out_specs=pl.BlockSpec((8, Z), lambda i: (0, 0)),
        out_shape=jax.ShapeDtypeStruct((8, Z), jnp.float32),
        scratch_shapes=[pltpu.VMEM((8, Z), jnp.float32),
                        pltpu.VMEM((8, Z), jnp.float32)],
    )(e2, e3, e4)


def _build_body(t2_ref, t3_ref, t4_ref, lse_ref, out_ref):
    i = pl.program_id(0)
    rows = lax.broadcasted_iota(jnp.int32, (VCHUNK, 1), 0) + i * VCHUNK
    for n, blk in enumerate((t2_ref, t3_ref, t4_ref)):
        v = blk[...] - lse_ref[n:n + 1, :]
        v = jnp.where(rows == 0, 0.0, v)
        v = jnp.where(rows == 1, NEG_INF, v)
        out_ref[n, :, :] = v


def _build_call(e2, e3, e4, lse):
    return pl.pallas_call(
        _build_body,
        grid=(NBLK,),
        in_specs=[pl.BlockSpec((VCHUNK, Z), lambda i: (i, 0))] * 3
        + [pl.BlockSpec((8, Z), lambda i: (0, 0))],
        out_specs=pl.BlockSpec((3, VCHUNK, Z), lambda i: (0, i, 0)),
        out_shape=jax.ShapeDtypeStruct((3, V, Z), jnp.float32),
    )(e2, e3, e4, lse)


def _sc_body(tall_ref, g_ref, out_ref, idx_v, buf0, buf1, sem0, sem1):
    wid = lax.axis_index("s") * NC + lax.axis_index("c")
    pltpu.sync_copy(g_ref.at[wid], idx_v)
    obase = wid * PW_ROWS

    def gather(c, buf, sem):
        return pltpu.make_async_copy(
            tall_ref.at[idx_v.at[pl.ds(c * CH_ROWS, CH_ROWS)]], buf, sem)

    def write(c, buf):
        pltpu.sync_copy(buf, out_ref.at[pl.ds(obase + c * CH_ROWS, CH_ROWS)])

    gather(0, buf0, sem0).start()

    def pair(k, carry):
        c0 = 2 * k
        c1 = c0 + 1
        gather(c1, buf1, sem1).start()
        gather(c0, buf0, sem0).wait()
        write(c0, buf0)

        @pl.when(c0 + 2 < NCHUNK)
        def _():
            gather(c0 + 2, buf0, sem0).start()

        gather(c1, buf1, sem1).wait()
        write(c1, buf1)
        return carry

    lax.fori_loop(0, NCHUNK // 2, pair, 0)


def _sc_call(tall_flat, g3):
    mesh = plsc.VectorSubcoreMesh(core_axis_name="c", subcore_axis_name="s")
    fn = pl.kernel(
        _sc_body,
        out_type=jax.ShapeDtypeStruct((ROWS, Z), jnp.float32),
        mesh=mesh,
        scratch_types=[
            pltpu.VMEM((PW_ROWS,), jnp.int32),
            pltpu.VMEM((CH_ROWS, Z), jnp.float32),
            pltpu.VMEM((CH_ROWS, Z), jnp.float32),
            pltpu.SemaphoreType.DMA,
            pltpu.SemaphoreType.DMA,
        ],
        compiler_params=pltpu.CompilerParams(use_tc_tiling_on_sc=False),
    )
    return fn(tall_flat, g3)


def _asm_body(in_ref, out_ref):
    k = pl.program_id(1)

    @pl.when(k < 2)
    def _z():
        out_ref[...] = jnp.zeros_like(out_ref)

    @pl.when(k >= 2)
    def _t():
        out_ref[0, 0] = jnp.transpose(in_ref[0, 0])


def _asm_call(gathered):
    return pl.pallas_call(
        _asm_body,
        grid=(S, K),
        in_specs=[pl.BlockSpec((1, 1, B, Z),
                               lambda s, k: (s, jnp.maximum(k - 2, 0), 0, 0))],
        out_specs=pl.BlockSpec((1, 1, Z, B), lambda s, k: (s, k, 0, 0)),
        out_shape=jax.ShapeDtypeStruct((S, K, Z, B), jnp.float32),
    )(gathered)


def kernel(x, x_lengths, subseq_ids_2, subseq_ids_3, subseq_ids_4,
           transition_matrix_z_z, length_emission_matrix_z_n,
           emission_table_2, emission_table_3, emission_table_4):
    lse = _lse_call(emission_table_2, emission_table_3, emission_table_4)
    tall = _build_call(emission_table_2, emission_table_3, emission_table_4, lse)
    tall_flat = tall.reshape(3 * V, Z)

    def clamp(ids):
        ids = ids.astype(jnp.int32)
        return jnp.where(ids >= V, UNK, ids)

    i2 = clamp(subseq_ids_2).T
    i3 = clamp(subseq_ids_3).T + V
    i4 = clamp(subseq_ids_4).T + 2 * V
    g3 = jnp.stack([i2, i3, i4], axis=1).reshape(W, PW_ROWS)

    gathered = _sc_call(tall_flat, g3).reshape(S, NSLOT, B, Z)
    out_phys = _asm_call(gathered)
    return out_phys.transpose(3, 0, 1, 2)


# speedup vs baseline: 8.3371x; 1.2784x over previous
import jax
import jax.numpy as jnp
from jax import lax
from jax.experimental import pallas as pl
from jax.experimental.pallas import tpu as pltpu
from jax.experimental.pallas import tpu_sc as plsc

Z = 64
V = 100000
B = 1024
S = 200
K = 5
NEG_INF = -1000000.0
UNK = 1

VCHUNK = 1000
NBLK = V // VCHUNK

NSLOT = 3
ROWS = S * NSLOT * B

try:
    _info = plsc.get_sparse_core_info()
    NC, NS = int(_info.num_cores), int(_info.num_subcores)
except Exception:
    NC, NS = 2, 16
W = NC * NS
CH_ROWS = 640
PW_ROWS = ROWS // W
NCHUNK = PW_ROWS // CH_ROWS


def _lse_body(t2_ref, t3_ref, t4_ref, out_ref, m_ref, s_ref):
    i = pl.program_id(0)
    last = pl.num_programs(0) - 1

    @pl.when(i == 0)
    def _init():
        m_ref[...] = jnp.zeros_like(m_ref)
        s_ref[...] = jnp.zeros_like(s_ref)

    for n, blk in enumerate((t2_ref, t3_ref, t4_ref)):
        x = blk[...]
        bm = jnp.max(x, axis=0, keepdims=True)
        bs = jnp.sum(jnp.exp(x - bm), axis=0, keepdims=True)
        m0 = m_ref[n:n + 1, :]
        s0 = s_ref[n:n + 1, :]
        mn = jnp.maximum(m0, bm)
        m_ref[n:n + 1, :] = mn
        s_ref[n:n + 1, :] = s0 * jnp.exp(m0 - mn) + bs * jnp.exp(bm - mn)

    @pl.when(i == last)
    def _fin():
        out_ref[...] = m_ref[...] + jnp.log(s_ref[...])


def _lse_call(e2, e3, e4):
    return pl.pallas_call(
        _lse_body,
        grid=(NBLK,),
        in_specs=[pl.BlockSpec((VCHUNK, Z), lambda i: (i, 0))] * 3,
        out_specs=pl.BlockSpec((8, Z), lambda i: (0, 0)),
        out_shape=jax.ShapeDtypeStruct((8, Z), jnp.float32),
        scratch_shapes=[pltpu.VMEM((8, Z), jnp.float32),
                        pltpu.VMEM((8, Z), jnp.float32)],
    )(e2, e3, e4)


def _build_body(t2_ref, t3_ref, t4_ref, lse_ref, out_ref):
    i = pl.program_id(0)
    rows = lax.broadcasted_iota(jnp.int32, (VCHUNK, 1), 0) + i * VCHUNK
    for n, blk in enumerate((t2_ref, t3_ref, t4_ref)):
        v = blk[...] - lse_ref[n:n + 1, :]
        v = jnp.where(rows == 0, 0.0, v)
        v = jnp.where(rows == 1, NEG_INF, v)
        out_ref[n, :, :] = v


def _build_call(e2, e3, e4, lse):
    return pl.pallas_call(
        _build_body,
        grid=(NBLK,),
        in_specs=[pl.BlockSpec((VCHUNK, Z), lambda i: (i, 0))] * 3
        + [pl.BlockSpec((8, Z), lambda i: (0, 0))],
        out_specs=pl.BlockSpec((3, VCHUNK, Z), lambda i: (0, i, 0)),
        out_shape=jax.ShapeDtypeStruct((3, V, Z), jnp.float32),
    )(e2, e3, e4, lse)


def _sc_body(tall_ref, g_ref, out_ref, idx_v, buf0, buf1, sem0, sem1):
    wid = lax.axis_index("s") * NC + lax.axis_index("c")
    pltpu.sync_copy(g_ref.at[wid], idx_v)
    obase = wid * PW_ROWS

    def gather(c, buf, sem):
        return pltpu.make_async_copy(
            tall_ref.at[idx_v.at[pl.ds(c * CH_ROWS, CH_ROWS)]], buf, sem)

    def write(c, buf):
        pltpu.sync_copy(buf, out_ref.at[pl.ds(obase + c * CH_ROWS, CH_ROWS)])

    gather(0, buf0, sem0).start()

    def pair(k, carry):
        c0 = 2 * k
        c1 = c0 + 1
        gather(c1, buf1, sem1).start()
        gather(c0, buf0, sem0).wait()
        write(c0, buf0)

        @pl.when(c0 + 2 < NCHUNK)
        def _():
            gather(c0 + 2, buf0, sem0).start()

        gather(c1, buf1, sem1).wait()
        write(c1, buf1)
        return carry

    lax.fori_loop(0, NCHUNK // 2, pair, 0)


def _sc_call(tall_flat, g3):
    mesh = plsc.VectorSubcoreMesh(core_axis_name="c", subcore_axis_name="s")
    fn = pl.kernel(
        _sc_body,
        out_type=jax.ShapeDtypeStruct((ROWS, Z), jnp.float32),
        mesh=mesh,
        scratch_types=[
            pltpu.VMEM((PW_ROWS,), jnp.int32),
            pltpu.VMEM((CH_ROWS, Z), jnp.float32),
            pltpu.VMEM((CH_ROWS, Z), jnp.float32),
            pltpu.SemaphoreType.DMA,
            pltpu.SemaphoreType.DMA,
        ],
        compiler_params=pltpu.CompilerParams(use_tc_tiling_on_sc=False),
    )
    return fn(tall_flat, g3)


def _asm_body(in_ref, out_ref):
    k = pl.program_id(0)

    @pl.when(k < 2)
    def _z():
        out_ref[...] = jnp.zeros_like(out_ref)

    @pl.when(k >= 2)
    def _t():
        x = in_ref[0, 0]
        xa = x[:, 0:Z]
        xb = x[:, Z:2 * Z]
        out_ref[0, 0] = jnp.concatenate(
            [jnp.transpose(xa), jnp.transpose(xb)], axis=1)


def _asm_call(gathered):
    return pl.pallas_call(
        _asm_body,
        grid=(K, S),
        in_specs=[pl.BlockSpec(
            (1, 1, B // 2, 2 * Z),
            lambda k, s: (jnp.where(k < 2, 0, s), jnp.maximum(k - 2, 0), 0, 0))],
        out_specs=pl.BlockSpec((1, 1, Z, B), lambda k, s: (s, k, 0, 0)),
        out_shape=jax.ShapeDtypeStruct((S, K, Z, B), jnp.float32),
    )(gathered)


def kernel(x, x_lengths, subseq_ids_2, subseq_ids_3, subseq_ids_4,
           transition_matrix_z_z, length_emission_matrix_z_n,
           emission_table_2, emission_table_3, emission_table_4):
    lse = _lse_call(emission_table_2, emission_table_3, emission_table_4)
    tall = _build_call(emission_table_2, emission_table_3, emission_table_4, lse)
    tall_flat = tall.reshape(3 * V, Z)

    def clamp(ids):
        ids = ids.astype(jnp.int32)
        return jnp.where(ids >= V, UNK, ids)

    def permuted(ids):
        return clamp(ids).reshape(2, B // 2, S).transpose(2, 1, 0).reshape(S, B)

    i2 = permuted(subseq_ids_2)
    i3 = permuted(subseq_ids_3) + V
    i4 = permuted(subseq_ids_4) + 2 * V
    g3 = jnp.stack([i2, i3, i4], axis=1).reshape(W, PW_ROWS)

    gathered = _sc_call(tall_flat, g3).reshape(S, NSLOT, B // 2, 2 * Z)
    out_phys = _asm_call(gathered)
    return out_phys.transpose(3, 0, 1, 2)


# speedup vs baseline: 10.3553x; 1.2421x over previous
import jax
import jax.numpy as jnp
from jax import lax
from jax.experimental import pallas as pl
from jax.experimental.pallas import tpu as pltpu
from jax.experimental.pallas import tpu_sc as plsc

Z = 64
V = 100000
B = 1024
S = 200
K = 5
NEG_INF = -1000000.0
UNK = 1

VCHUNK = 1000
NBLK = V // VCHUNK

NSLOT = 3
ROWS = S * NSLOT * B

try:
    _info = plsc.get_sparse_core_info()
    NC, NS = int(_info.num_cores), int(_info.num_subcores)
except Exception:
    NC, NS = 2, 16
W = NC * NS
CH_ROWS = 640
PW_ROWS = ROWS // W
NCHUNK = PW_ROWS // CH_ROWS


def _lse_body(t2_ref, t3_ref, t4_ref, out_ref, m_ref, s_ref):
    i = pl.program_id(0)
    last = pl.num_programs(0) - 1

    @pl.when(i == 0)
    def _init():
        m_ref[...] = jnp.zeros_like(m_ref)
        s_ref[...] = jnp.zeros_like(s_ref)

    for n, blk in enumerate((t2_ref, t3_ref, t4_ref)):
        x = blk[...]
        bm = jnp.max(x, axis=0, keepdims=True)
        bs = jnp.sum(jnp.exp(x - bm), axis=0, keepdims=True)
        m0 = m_ref[n:n + 1, :]
        s0 = s_ref[n:n + 1, :]
        mn = jnp.maximum(m0, bm)
        m_ref[n:n + 1, :] = mn
        s_ref[n:n + 1, :] = s0 * jnp.exp(m0 - mn) + bs * jnp.exp(bm - mn)

    @pl.when(i == last)
    def _fin():
        out_ref[...] = m_ref[...] + jnp.log(s_ref[...])


def _lse_call(e2, e3, e4):
    return pl.pallas_call(
        _lse_body,
        grid=(NBLK,),
        in_specs=[pl.BlockSpec((VCHUNK, Z), lambda i: (i, 0))] * 3,
        out_specs=pl.BlockSpec((8, Z), lambda i: (0, 0)),
        out_shape=jax.ShapeDtypeStruct((8, Z), jnp.float32),
        scratch_shapes=[pltpu.VMEM((8, Z), jnp.float32),
                        pltpu.VMEM((8, Z), jnp.float32)],
    )(e2, e3, e4)


def _build_body(t2_ref, t3_ref, t4_ref, lse_ref, out_ref):
    i = pl.program_id(0)
    rows = lax.broadcasted_iota(jnp.int32, (VCHUNK, 1), 0) + i * VCHUNK
    for n, blk in enumerate((t2_ref, t3_ref, t4_ref)):
        v = blk[...] - lse_ref[n:n + 1, :]
        v = jnp.where(rows == 0, 0.0, v)
        v = jnp.where(rows == 1, NEG_INF, v)
        out_ref[n, :, :] = v


def _build_call(e2, e3, e4, lse):
    return pl.pallas_call(
        _build_body,
        grid=(NBLK,),
        in_specs=[pl.BlockSpec((VCHUNK, Z), lambda i: (i, 0))] * 3
        + [pl.BlockSpec((8, Z), lambda i: (0, 0))],
        out_specs=pl.BlockSpec((3, VCHUNK, Z), lambda i: (0, i, 0)),
        out_shape=jax.ShapeDtypeStruct((3, V, Z), jnp.float32),
    )(e2, e3, e4, lse)


def _sc_body(tall_ref, g_ref, out_ref, idx_v, buf0, buf1, sem0, sem1):
    wid = lax.axis_index("s") * NC + lax.axis_index("c")
    pltpu.sync_copy(g_ref.at[wid], idx_v)
    obase = wid * PW_ROWS

    def gather(c, buf, sem):
        return pltpu.make_async_copy(
            tall_ref.at[idx_v.at[pl.ds(c * CH_ROWS, CH_ROWS)]], buf, sem)

    def write(c, buf):
        pltpu.sync_copy(buf, out_ref.at[pl.ds(obase + c * CH_ROWS, CH_ROWS)])

    gather(0, buf0, sem0).start()

    def pair(k, carry):
        c0 = 2 * k
        c1 = c0 + 1
        gather(c1, buf1, sem1).start()
        gather(c0, buf0, sem0).wait()
        write(c0, buf0)

        @pl.when(c0 + 2 < NCHUNK)
        def _():
            gather(c0 + 2, buf0, sem0).start()

        gather(c1, buf1, sem1).wait()
        write(c1, buf1)
        return carry

    lax.fori_loop(0, NCHUNK // 2, pair, 0)


def _sc_call(tall_flat, g3):
    mesh = plsc.VectorSubcoreMesh(core_axis_name="c", subcore_axis_name="s")
    fn = pl.kernel(
        _sc_body,
        out_type=jax.ShapeDtypeStruct((ROWS, Z), jnp.float32),
        mesh=mesh,
        scratch_types=[
            pltpu.VMEM((PW_ROWS,), jnp.int32),
            pltpu.VMEM((CH_ROWS, Z), jnp.float32),
            pltpu.VMEM((CH_ROWS, Z), jnp.float32),
            pltpu.SemaphoreType.DMA,
            pltpu.SemaphoreType.DMA,
        ],
        compiler_params=pltpu.CompilerParams(use_tc_tiling_on_sc=False),
    )
    return fn(tall_flat, g3)


SBLK = 2


def _asm_body(in_ref, out_ref):
    k = pl.program_id(0)

    @pl.when(k < 2)
    def _z():
        out_ref[...] = jnp.zeros_like(out_ref)

    @pl.when(k >= 2)
    def _t():
        ri = lax.broadcasted_iota(jnp.int32, (Z, Z), 0)
        ci = lax.broadcasted_iota(jnp.int32, (Z, Z), 1)
        ident = (ri == ci).astype(jnp.float32)
        for t in range(SBLK):
            x = in_ref[t, 0]
            ta = lax.dot_general(ident, x[:, 0:Z],
                                 (((1,), (1,)), ((), ())),
                                 preferred_element_type=jnp.float32)
            tb = lax.dot_general(ident, x[:, Z:2 * Z],
                                 (((1,), (1,)), ((), ())),
                                 preferred_element_type=jnp.float32)
            out_ref[t, 0] = jnp.concatenate([ta, tb], axis=1)


def _asm_call(gathered):
    return pl.pallas_call(
        _asm_body,
        grid=(K, S // SBLK),
        in_specs=[pl.BlockSpec(
            (SBLK, 1, B // 2, 2 * Z),
            lambda k, s: (jnp.where(k < 2, 0, s), jnp.maximum(k - 2, 0), 0, 0))],
        out_specs=pl.BlockSpec((SBLK, 1, Z, B), lambda k, s: (s, k, 0, 0)),
        out_shape=jax.ShapeDtypeStruct((S, K, Z, B), jnp.float32),
    )(gathered)


def kernel(x, x_lengths, subseq_ids_2, subseq_ids_3, subseq_ids_4,
           transition_matrix_z_z, length_emission_matrix_z_n,
           emission_table_2, emission_table_3, emission_table_4):
    lse = _lse_call(emission_table_2, emission_table_3, emission_table_4)
    tall = _build_call(emission_table_2, emission_table_3, emission_table_4, lse)
    tall_flat = tall.reshape(3 * V, Z)

    def clamp(ids):
        ids = ids.astype(jnp.int32)
        return jnp.where(ids >= V, UNK, ids)

    def permuted(ids):
        return clamp(ids).reshape(2, B // 2, S).transpose(2, 1, 0).reshape(S, B)

    i2 = permuted(subseq_ids_2)
    i3 = permuted(subseq_ids_3) + V
    i4 = permuted(subseq_ids_4) + 2 * V
    g3 = jnp.stack([i2, i3, i4], axis=1).reshape(W, PW_ROWS)

    gathered = _sc_call(tall_flat, g3).reshape(S, NSLOT, B // 2, 2 * Z)
    out_phys = _asm_call(gathered)
    return out_phys.transpose(3, 0, 1, 2)
